# baseline (speedup 1.0000x reference)
import functools

import jax
import jax.numpy as jnp
from jax.experimental import pallas as pl


def _fm_block(attrs_ref, emb_ref, w_ref, out_ref):
    ae = jnp.dot(attrs_ref[...], w_ref[...], preferred_element_type=jnp.float32)
    out_ref[...] = jnp.concatenate([emb_ref[...], ae], axis=1)


@functools.partial(jax.jit, static_argnames=("block_rows",))
def _fm(attrs, emb, w, block_rows):
    n, d_emb = emb.shape
    d_attr = attrs.shape[1]
    d_out = d_emb + w.shape[1]
    grid = (n // block_rows,)
    return pl.pallas_call(
        _fm_block,
        grid=grid,
        in_specs=[
            pl.BlockSpec((block_rows, d_attr), lambda i: (i, 0)),
            pl.BlockSpec((block_rows, d_emb), lambda i: (i, 0)),
            pl.BlockSpec((d_attr, w.shape[1]), lambda i: (0, 0)),
        ],
        out_specs=pl.BlockSpec((block_rows, d_out), lambda i: (i, 0)),
        out_shape=jax.ShapeDtypeStruct((n, d_out), jnp.float32),
    )(attrs, emb, w)


def kernel(user_attrs_input, item_attrs_input, user_emb, item_emb,
           user_attrs_trans_w, item_attrs_trans_w):
    final_user = _fm(user_attrs_input, user_emb, user_attrs_trans_w, 10000)
    final_item = _fm(item_attrs_input, item_emb, item_attrs_trans_w, 10000)
    return (final_user, final_item)
